# baseline (speedup 1.0000x reference)
import functools

import numpy as np
import jax
import jax.numpy as jnp
from jax import lax
from jax.experimental import pallas as pl
from jax.experimental.pallas import tpu as pltpu
from jax.experimental.pallas import tpu_sc as plsc

_B = 16384
_F = 26
_D = 16
_TOTAL = _B * _F
_NW = 32
_IDX_ROWS = _TOTAL // 128
_ROWS_PER_TILE = _TOTAL // _NW
_IROWS_PER_TILE = _IDX_ROWS // _NW
_GROUPS = 8
_K = _IROWS_PER_TILE // _GROUPS
_GROUP_ROWS = _K * 128

_OFFS = jnp.asarray(np.arange(_F, dtype=np.int32) * 100000)

_P_NP = np.zeros((_F * _D, _D), dtype=np.float32)
_P_NP[np.arange(_F * _D), np.arange(_F * _D) % _D] = 1.0

_BN_C = float(1.0 / np.sqrt(1.0 + 1e-5))


@functools.cache
def _make_sc_gather():
    mesh = plsc.VectorSubcoreMesh(core_axis_name="c", subcore_axis_name="s")
    return functools.partial(
        pl.kernel,
        mesh=mesh,
        out_type=[
            jax.ShapeDtypeStruct((_TOTAL, _D), jnp.float32),
            jax.ShapeDtypeStruct((_IDX_ROWS, 128), jnp.float32),
        ],
        scratch_types=[
            pltpu.VMEM((_IROWS_PER_TILE, 128), jnp.int32),
            pltpu.VMEM((_IROWS_PER_TILE, 128), jnp.float32),
            pltpu.VMEM((_GROUP_ROWS, _D), jnp.float32),
            pltpu.SemaphoreType.DMA,
            pltpu.SemaphoreType.DMA,
        ],
        compiler_params=pltpu.CompilerParams(use_tc_tiling_on_sc=False),
    )(_sc_gather_body)


def _sc_gather_body(table, lr_flat, idx, emb_out, lr_out, idx_v, lr_v, buf, sem_e, sem_l):
    wid = lax.axis_index("s") * 2 + lax.axis_index("c")
    gbase = wid * _IROWS_PER_TILE
    pltpu.sync_copy(idx.at[pl.ds(gbase, _IROWS_PER_TILE)], idx_v)

    def group(g, carry):
        copies = []
        for j in range(_K):
            r = g * _K + j
            copies.append(pltpu.async_copy(
                table.at[idx_v.at[r]], buf.at[pl.ds(j * 128, 128)], sem_e))
            copies.append(pltpu.async_copy(
                lr_flat.at[idx_v.at[r]], lr_v.at[r], sem_l))
        for c in copies:
            c.wait()
        pltpu.sync_copy(
            buf,
            emb_out.at[pl.ds(wid * _ROWS_PER_TILE + g * _GROUP_ROWS, _GROUP_ROWS)])
        return carry

    lax.fori_loop(0, _GROUPS, group, 0)
    pltpu.sync_copy(lr_v, lr_out.at[pl.ds(gbase, _IROWS_PER_TILE)])


def _mlp_body(emb_ref, lrv_ref, W1_ref, b1_ref, W2_ref, b2_ref, W3_ref,
              b3_ref, Wo_ref, bias_ref, P_ref, o_ref):
    h0 = emb_ref[...]
    s = jnp.dot(h0, P_ref[...], preferred_element_type=jnp.float32)
    sq_sum = jnp.sum(h0 * h0, axis=1, keepdims=True)
    fm = 0.5 * (jnp.sum(s * s, axis=1, keepdims=True) - sq_sum)
    lr_sum = jnp.sum(lrv_ref[...], axis=1, keepdims=True)
    h = jnp.maximum(
        (jnp.dot(h0, W1_ref[...], preferred_element_type=jnp.float32)
         + b1_ref[...]) * _BN_C, 0.0)
    h = jnp.maximum(
        (jnp.dot(h, W2_ref[...], preferred_element_type=jnp.float32)
         + b2_ref[...]) * _BN_C, 0.0)
    h = jnp.maximum(
        (jnp.dot(h, W3_ref[...], preferred_element_type=jnp.float32)
         + b3_ref[...]) * _BN_C, 0.0)
    mlp = jnp.dot(h, Wo_ref[...], preferred_element_type=jnp.float32)
    o_ref[...] = lr_sum + fm + mlp + bias_ref[...]


def _tc_mlp(emb, lrv, W1, b1, W2, b2, W3, b3, Wo, bias, P, bs):
    n = _B // bs
    full = lambda shape: pl.BlockSpec(shape, lambda i: (0, 0))
    return pl.pallas_call(
        _mlp_body,
        grid=(n,),
        in_specs=[
            pl.BlockSpec((bs, _F * _D), lambda i: (i, 0)),
            pl.BlockSpec((bs, _F), lambda i: (i, 0)),
            full(W1.shape), full((1, 400)),
            full(W2.shape), full((1, 400)),
            full(W3.shape), full((1, 400)),
            full(Wo.shape), full((1, 1)), full(P.shape),
        ],
        out_specs=pl.BlockSpec((bs, 1), lambda i: (i, 0)),
        out_shape=jax.ShapeDtypeStruct((_B, 1), jnp.float32),
        compiler_params=pltpu.CompilerParams(
            dimension_semantics=("arbitrary",)),
    )(emb, lrv, W1, b1, W2, b2, W3, b3, Wo, bias, P)


def kernel(x, emb_table, lr_table, lr_bias, W1, b1, W2, b2, W3, b3, Wo, bo):
    xo = (x + _OFFS[None, :]).reshape(_IDX_ROWS, 128)
    emb_flat, lr_vals = _make_sc_gather()(emb_table, lr_table.reshape(-1), xo)
    emb2 = emb_flat.reshape(_B, _F * _D)
    lrv = lr_vals.reshape(_B, _F)
    bias = (lr_bias + bo).reshape(1, 1)
    P = jnp.asarray(_P_NP)
    return _tc_mlp(emb2, lrv, W1, b1.reshape(1, -1), W2, b2.reshape(1, -1),
                   W3, b3.reshape(1, -1), Wo, bias, P, bs=1024)
